# Initial kernel scaffold; baseline (speedup 1.0000x reference)
import jax
import jax.numpy as jnp
from jax.experimental import pallas as pl


def kernel(x, edge_index, W1, b1, W2, b2):
    raise NotImplementedError("write your pallas kernel here")


# speedup vs baseline: 14.6577x; 14.6577x over previous
import functools

import jax
import jax.numpy as jnp
from jax import lax
from jax.experimental import pallas as pl
from jax.experimental.pallas import tpu as pltpu
from jax.experimental.pallas import tpu_sc as plsc

N = 10000
E = 320000
IN_DIM = 128
HID = 1000
OUT_DIM = 128

NC = 2
NS = 16
NW = NC * NS
CHUNK = 128
CPW = -(-E // (NW * CHUNK))
E_PAD = NW * CPW * CHUNK
NP = 10112
RPT = NP // NS

@functools.cache
def _mesh():
    return plsc.VectorSubcoreMesh(
        core_axis_name="c", subcore_axis_name="s",
        num_cores=NC, num_subcores=NS)


def _sc_deg_body(dst_hbm, ones_hbm, zero_hbm, out_hbm, dacc, didx, ones_v):
    c = lax.axis_index("c")
    s = lax.axis_index("s")
    wid = s * NC + c
    pltpu.sync_copy(zero_hbm.at[pl.ds(s * RPT, RPT)], dacc.at[pl.ds(s * RPT, RPT)])
    pltpu.sync_copy(ones_hbm, ones_v)
    pltpu.sync_copy(dst_hbm.at[wid], didx)
    plsc.subcore_barrier()

    def step(i, carry):
        pltpu.sync_copy(ones_v, dacc.at[didx.at[i]], add=True)
        return carry

    lax.fori_loop(0, CPW, step, 0)
    plsc.subcore_barrier()
    pltpu.sync_copy(dacc.at[pl.ds(s * RPT, RPT)], out_hbm.at[c, pl.ds(s * RPT, RPT)])


@functools.cache
def _sc_deg_kernel():
    return pl.kernel(
        _sc_deg_body,
        out_type=jax.ShapeDtypeStruct((NC, NP, IN_DIM), jnp.float32),
        mesh=_mesh(),
        scratch_types=[
            pltpu.VMEM_SHARED((NP, IN_DIM), jnp.float32),
            pltpu.VMEM((CPW, CHUNK), jnp.int32),
            pltpu.VMEM((CHUNK, IN_DIM), jnp.float32),
        ],
    )


def _sc_deg(dst3, ones128, zeros128):
    return _sc_deg_kernel()(dst3, ones128, zeros128)


def _sc_agg_body(y_hbm, src_hbm, dst_hbm, zero_hbm, out_hbm, zacc, sidx, didx, rows):
    c = lax.axis_index("c")
    s = lax.axis_index("s")
    wid = s * NC + c
    pltpu.sync_copy(zero_hbm.at[pl.ds(s * RPT, RPT)], zacc.at[pl.ds(s * RPT, RPT)])
    pltpu.sync_copy(src_hbm.at[wid], sidx)
    pltpu.sync_copy(dst_hbm.at[wid], didx)
    plsc.subcore_barrier()

    def step(i, carry):
        pltpu.sync_copy(y_hbm.at[sidx.at[i]], rows)
        pltpu.sync_copy(rows, zacc.at[didx.at[i]], add=True)
        return carry

    lax.fori_loop(0, CPW, step, 0)
    plsc.subcore_barrier()
    pltpu.sync_copy(zacc.at[pl.ds(s * RPT, RPT)], out_hbm.at[c, pl.ds(s * RPT, RPT)])


@functools.cache
def _sc_agg_kernel():
    return pl.kernel(
        _sc_agg_body,
        out_type=jax.ShapeDtypeStruct((NC, NP, IN_DIM), jnp.float32),
        mesh=_mesh(),
        scratch_types=[
            pltpu.VMEM_SHARED((NP, IN_DIM), jnp.float32),
            pltpu.VMEM((CPW, CHUNK), jnp.int32),
            pltpu.VMEM((CPW, CHUNK), jnp.int32),
            pltpu.VMEM((CHUNK, IN_DIM), jnp.float32),
        ],
    )


def _sc_agg(y_ext, src3, dst3, zeros128):
    return _sc_agg_kernel()(y_ext, src3, dst3, zeros128)


BLK = 1024
GRID = -(-NP // BLK)
GRID_OUT = -(-N // BLK)


def _tc_prep_body(deg_ref, x_ref, y_ref, dinv_ref):
    i = pl.program_id(0)
    deg = deg_ref[0] + deg_ref[1] + 1.0
    row = i * BLK + lax.broadcasted_iota(jnp.int32, (BLK, IN_DIM), 0)
    dinv = jnp.where(row < N, lax.rsqrt(deg), 0.0)
    dinv_ref[...] = dinv
    y_ref[...] = x_ref[...] * dinv


def _tc_prep(deg_b, x_ext):
    return pl.pallas_call(
        _tc_prep_body,
        grid=(GRID,),
        in_specs=[
            pl.BlockSpec((NC, BLK, IN_DIM), lambda i: (0, i, 0)),
            pl.BlockSpec((BLK, IN_DIM), lambda i: (i, 0)),
        ],
        out_specs=[
            pl.BlockSpec((BLK, IN_DIM), lambda i: (i, 0)),
            pl.BlockSpec((BLK, IN_DIM), lambda i: (i, 0)),
        ],
        out_shape=[
            jax.ShapeDtypeStruct((NP, IN_DIM), jnp.float32),
            jax.ShapeDtypeStruct((NP, IN_DIM), jnp.float32),
        ],
    )(deg_b, x_ext)


def _tc_mid_body(z1_ref, y_ref, dinv_ref, w1_ref, b1_ref, w2_ref, y2_ref):
    a = dinv_ref[...] * (z1_ref[0] + z1_ref[1] + y_ref[...])
    h = jnp.dot(a, w1_ref[...], preferred_element_type=jnp.float32) + b1_ref[...]
    h = jnp.maximum(h, 0.0)
    y2_ref[...] = dinv_ref[...] * jnp.dot(
        h, w2_ref[...], preferred_element_type=jnp.float32)


def _tc_mid(z1, y_ext, dinv_b, w1, b1, w2):
    return pl.pallas_call(
        _tc_mid_body,
        grid=(GRID,),
        in_specs=[
            pl.BlockSpec((NC, BLK, IN_DIM), lambda i: (0, i, 0)),
            pl.BlockSpec((BLK, IN_DIM), lambda i: (i, 0)),
            pl.BlockSpec((BLK, IN_DIM), lambda i: (i, 0)),
            pl.BlockSpec((IN_DIM, HID), lambda i: (0, 0)),
            pl.BlockSpec((1, HID), lambda i: (0, 0)),
            pl.BlockSpec((HID, OUT_DIM), lambda i: (0, 0)),
        ],
        out_specs=pl.BlockSpec((BLK, IN_DIM), lambda i: (i, 0)),
        out_shape=jax.ShapeDtypeStruct((NP, IN_DIM), jnp.float32),
    )(z1, y_ext, dinv_b, w1, b1, w2)


def _tc_fin_body(z2_ref, y2_ref, dinv_ref, b2_ref, o_ref):
    o_ref[...] = (dinv_ref[...] * (z2_ref[0] + z2_ref[1] + y2_ref[...])
                  + b2_ref[...])


def _tc_fin(z2, y2_ext, dinv_b, b2):
    return pl.pallas_call(
        _tc_fin_body,
        grid=(GRID_OUT,),
        in_specs=[
            pl.BlockSpec((NC, BLK, OUT_DIM), lambda i: (0, i, 0)),
            pl.BlockSpec((BLK, OUT_DIM), lambda i: (i, 0)),
            pl.BlockSpec((BLK, OUT_DIM), lambda i: (i, 0)),
            pl.BlockSpec((1, OUT_DIM), lambda i: (0, 0)),
        ],
        out_specs=pl.BlockSpec((BLK, OUT_DIM), lambda i: (i, 0)),
        out_shape=jax.ShapeDtypeStruct((N, OUT_DIM), jnp.float32),
    )(z2, y2_ext, dinv_b, b2)


def kernel(x, edge_index, W1, b1, W2, b2):
    src = edge_index[0].astype(jnp.int32)
    dst = edge_index[1].astype(jnp.int32)
    pad = jnp.full((E_PAD - E,), N, dtype=jnp.int32)
    src3 = jnp.concatenate([src, pad]).reshape(NW, CPW, CHUNK)
    dst3 = jnp.concatenate([dst, pad]).reshape(NW, CPW, CHUNK)
    x_ext = jnp.concatenate(
        [x, jnp.zeros((NP - N, IN_DIM), dtype=jnp.float32)], axis=0)

    ones128 = jnp.ones((CHUNK, IN_DIM), dtype=jnp.float32)
    zeros128 = jnp.zeros((NP, IN_DIM), dtype=jnp.float32)

    deg_b = _sc_deg(dst3, ones128, zeros128)
    y_ext, dinv_b = _tc_prep(deg_b, x_ext)
    z1 = _sc_agg(y_ext, src3, dst3, zeros128)
    y2_ext = _tc_mid(z1, y_ext, dinv_b, W1, b1.reshape(1, HID), W2)
    z2 = _sc_agg(y2_ext, src3, dst3, zeros128)
    return _tc_fin(z2, y2_ext, dinv_b, b2.reshape(1, OUT_DIM))


# speedup vs baseline: 38.5659x; 2.6311x over previous
import functools

import jax
import jax.numpy as jnp
from jax import lax
from jax.experimental import pallas as pl
from jax.experimental.pallas import tpu as pltpu
from jax.experimental.pallas import tpu_sc as plsc

N = 10000
E = 320000
IN_DIM = 128
HID = 1000
OUT_DIM = 128

NC = 2
NS = 16
NW = NC * NS
CHUNK = 128
CPW = 2 * (-(-E // (NW * CHUNK * 2)))
TOTC = NW * CPW
STAGE = CPW // 2
E_PAD = NW * CPW * CHUNK
NP = 10240
RPT = NP // NS

@functools.cache
def _mesh():
    return plsc.VectorSubcoreMesh(
        core_axis_name="c", subcore_axis_name="s",
        num_cores=NC, num_subcores=NS)


def _sc_deg_body(dst_hbm, ones_hbm, zero_hbm, out_hbm,
                 dacc, didx, ones_v, sem0, sem1):
    c = lax.axis_index("c")
    s = lax.axis_index("s")
    wid = s * NC + c
    pltpu.sync_copy(zero_hbm.at[pl.ds(s * RPT, RPT)], dacc.at[pl.ds(s * RPT, RPT)])
    pltpu.sync_copy(ones_hbm, ones_v)
    pltpu.sync_copy(dst_hbm.at[pl.ds(wid * CPW, CPW)], didx)
    plsc.subcore_barrier()

    pltpu.async_copy(ones_v, dacc.at[didx.at[0]], sem0, add=True)

    def step(j, carry):
        i = 2 * j
        pltpu.async_copy(ones_v, dacc.at[didx.at[i + 1]], sem1, add=True)
        pltpu.make_async_copy(ones_v, dacc.at[didx.at[i]], sem0).wait()

        @pl.when(j < CPW // 2 - 1)
        def _():
            pltpu.async_copy(ones_v, dacc.at[didx.at[i + 2]], sem0, add=True)

        pltpu.make_async_copy(ones_v, dacc.at[didx.at[i + 1]], sem1).wait()
        return carry

    lax.fori_loop(0, CPW // 2, step, 0)
    plsc.subcore_barrier()
    pltpu.sync_copy(dacc.at[pl.ds(s * RPT, RPT)],
                    out_hbm.at[pl.ds(c * NP + s * RPT, RPT)])


@functools.cache
def _sc_deg_kernel():
    return pl.kernel(
        _sc_deg_body,
        out_type=jax.ShapeDtypeStruct((NC * NP,), jnp.float32),
        mesh=_mesh(),
        scratch_types=[
            pltpu.VMEM_SHARED((NP,), jnp.float32),
            pltpu.VMEM((CPW, CHUNK), jnp.int32),
            pltpu.VMEM((CHUNK,), jnp.float32),
            pltpu.SemaphoreType.DMA,
            pltpu.SemaphoreType.DMA,
        ],
    )


def _sc_deg(dst3, ones1, zeros1):
    return _sc_deg_kernel()(dst3, ones1, zeros1)


def _sc_agg_body(y_hbm, src_hbm, dst_hbm, zero_hbm, out_hbm,
                 zacc, sidx, didx, rows0, rows1, sem0, sem1):
    c = lax.axis_index("c")
    s = lax.axis_index("s")
    wid = s * NC + c
    base = wid * CPW
    pltpu.sync_copy(zero_hbm.at[pl.ds(s * RPT, RPT)], zacc.at[pl.ds(s * RPT, RPT)])
    plsc.subcore_barrier()

    def stage_loop(hs, carry0):
        pltpu.sync_copy(src_hbm.at[pl.ds(base + hs * STAGE, STAGE)], sidx)
        pltpu.sync_copy(dst_hbm.at[pl.ds(base + hs * STAGE, STAGE)], didx)
        pltpu.async_copy(y_hbm.at[sidx.at[0]], rows0, sem0)

        def step(j, carry):
            i = 2 * j
            pltpu.async_copy(y_hbm.at[sidx.at[i + 1]], rows1, sem1)
            pltpu.make_async_copy(y_hbm.at[sidx.at[i]], rows0, sem0).wait()
            pltpu.sync_copy(rows0, zacc.at[didx.at[i]], add=True)

            @pl.when(j < STAGE // 2 - 1)
            def _():
                pltpu.async_copy(y_hbm.at[sidx.at[i + 2]], rows0, sem0)

            pltpu.make_async_copy(y_hbm.at[sidx.at[i + 1]], rows1, sem1).wait()
            pltpu.sync_copy(rows1, zacc.at[didx.at[i + 1]], add=True)
            return carry

        lax.fori_loop(0, STAGE // 2, step, carry0)
        return carry0

    lax.fori_loop(0, CPW // STAGE, stage_loop, 0)
    plsc.subcore_barrier()
    pltpu.sync_copy(zacc.at[pl.ds(s * RPT, RPT)], out_hbm.at[c, pl.ds(s * RPT, RPT)])


@functools.cache
def _sc_agg_kernel():
    return pl.kernel(
        _sc_agg_body,
        out_type=jax.ShapeDtypeStruct((NC, NP, IN_DIM), jnp.float32),
        mesh=_mesh(),
        scratch_types=[
            pltpu.VMEM_SHARED((NP, IN_DIM), jnp.float32),
            pltpu.VMEM((STAGE, CHUNK), jnp.int32),
            pltpu.VMEM((STAGE, CHUNK), jnp.int32),
            pltpu.VMEM((CHUNK, IN_DIM), jnp.float32),
            pltpu.VMEM((CHUNK, IN_DIM), jnp.float32),
            pltpu.SemaphoreType.DMA,
            pltpu.SemaphoreType.DMA,
        ],
    )


def _sc_agg(y_ext, src3, dst3, zeros128):
    return _sc_agg_kernel()(y_ext, src3, dst3, zeros128)


BLK = 1024
GRID = -(-NP // BLK)
GRID_OUT = -(-N // BLK)


def _tc_prep_body(deg_ref, x_ref, y_ref, dinv_ref):
    i = pl.program_id(0)
    deg = deg_ref[...] + 1.0
    row = i * BLK + lax.broadcasted_iota(jnp.int32, (BLK, IN_DIM), 0)
    dinv = jnp.where(row < N, lax.rsqrt(deg), 0.0)
    dinv_ref[...] = dinv
    y_ref[...] = x_ref[...] * dinv


def _tc_prep(deg_b, x_ext):
    return pl.pallas_call(
        _tc_prep_body,
        grid=(GRID,),
        in_specs=[
            pl.BlockSpec((BLK, IN_DIM), lambda i: (i, 0)),
            pl.BlockSpec((BLK, IN_DIM), lambda i: (i, 0)),
        ],
        out_specs=[
            pl.BlockSpec((BLK, IN_DIM), lambda i: (i, 0)),
            pl.BlockSpec((BLK, IN_DIM), lambda i: (i, 0)),
        ],
        out_shape=[
            jax.ShapeDtypeStruct((NP, IN_DIM), jnp.float32),
            jax.ShapeDtypeStruct((NP, IN_DIM), jnp.float32),
        ],
    )(deg_b, x_ext)


def _tc_mid_body(z1_ref, y_ref, dinv_ref, w1_ref, b1_ref, w2_ref, y2_ref):
    a = dinv_ref[...] * (z1_ref[0] + z1_ref[1] + y_ref[...])
    h = jnp.dot(a, w1_ref[...], preferred_element_type=jnp.float32) + b1_ref[...]
    h = jnp.maximum(h, 0.0)
    y2_ref[...] = dinv_ref[...] * jnp.dot(
        h, w2_ref[...], preferred_element_type=jnp.float32)


def _tc_mid(z1, y_ext, dinv_b, w1, b1, w2):
    return pl.pallas_call(
        _tc_mid_body,
        grid=(GRID,),
        in_specs=[
            pl.BlockSpec((NC, BLK, IN_DIM), lambda i: (0, i, 0)),
            pl.BlockSpec((BLK, IN_DIM), lambda i: (i, 0)),
            pl.BlockSpec((BLK, IN_DIM), lambda i: (i, 0)),
            pl.BlockSpec((IN_DIM, HID), lambda i: (0, 0)),
            pl.BlockSpec((1, HID), lambda i: (0, 0)),
            pl.BlockSpec((HID, OUT_DIM), lambda i: (0, 0)),
        ],
        out_specs=pl.BlockSpec((BLK, IN_DIM), lambda i: (i, 0)),
        out_shape=jax.ShapeDtypeStruct((NP, IN_DIM), jnp.float32),
    )(z1, y_ext, dinv_b, w1, b1, w2)


def _tc_fin_body(z2_ref, y2_ref, dinv_ref, b2_ref, o_ref):
    o_ref[...] = (dinv_ref[...] * (z2_ref[0] + z2_ref[1] + y2_ref[...])
                  + b2_ref[...])


def _tc_fin(z2, y2_ext, dinv_b, b2):
    return pl.pallas_call(
        _tc_fin_body,
        grid=(GRID_OUT,),
        in_specs=[
            pl.BlockSpec((NC, BLK, OUT_DIM), lambda i: (0, i, 0)),
            pl.BlockSpec((BLK, OUT_DIM), lambda i: (i, 0)),
            pl.BlockSpec((BLK, OUT_DIM), lambda i: (i, 0)),
            pl.BlockSpec((1, OUT_DIM), lambda i: (0, 0)),
        ],
        out_specs=pl.BlockSpec((BLK, OUT_DIM), lambda i: (i, 0)),
        out_shape=jax.ShapeDtypeStruct((N, OUT_DIM), jnp.float32),
    )(z2, y2_ext, dinv_b, b2)


def kernel(x, edge_index, W1, b1, W2, b2):
    src = edge_index[0].astype(jnp.int32)
    dst = edge_index[1].astype(jnp.int32)
    pad = N + (jnp.arange(E_PAD - E, dtype=jnp.int32) % (NP - N))
    src3 = jnp.concatenate([src, pad]).reshape(TOTC, CHUNK)
    dst3 = jnp.concatenate([dst, pad]).reshape(TOTC, CHUNK)
    x_ext = jnp.concatenate(
        [x, jnp.zeros((NP - N, IN_DIM), dtype=jnp.float32)], axis=0)

    ones1 = jnp.ones((CHUNK,), dtype=jnp.float32)
    zeros1 = jnp.zeros((NP,), dtype=jnp.float32)
    zeros128 = jnp.zeros((NP, IN_DIM), dtype=jnp.float32)

    deg1 = _sc_deg(dst3, ones1, zeros1)
    degsum = deg1.reshape(NC, NP).sum(axis=0)
    deg_b = jnp.broadcast_to(degsum[:, None], (NP, IN_DIM))
    y_ext, dinv_b = _tc_prep(deg_b, x_ext)
    z1 = _sc_agg(y_ext, src3, dst3, zeros128)
    y2_ext = _tc_mid(z1, y_ext, dinv_b, W1, b1.reshape(1, HID), W2)
    z2 = _sc_agg(y2_ext, src3, dst3, zeros128)
    return _tc_fin(z2, y2_ext, dinv_b, b2.reshape(1, OUT_DIM))
